# baseline (speedup 1.0000x reference)
import functools

import jax
import jax.numpy as jnp
from jax import lax
from jax.experimental import pallas as pl
from jax.experimental.pallas import tpu as pltpu
from jax.experimental.pallas import tpu_sc as plsc

CH = 128
LANES = 16
NW = 32
CPW = 81
NBUF = 3


@functools.lru_cache(maxsize=None)
def _make_sc_aggregate(n_nodes: int, d: int):
    rows_per_tile = (n_nodes // (16 * 8)) * 8
    hop = 104
    n_hops = rows_per_tile // hop
    assert n_hops * hop == rows_per_tile
    tail = n_nodes - 16 * rows_per_tile
    assert 0 <= tail <= CH and tail % 8 == 0
    vregs_per_row = d // LANES

    mesh = plsc.VectorSubcoreMesh(core_axis_name="c", subcore_axis_name="s")

    rows_t = pltpu.VMEM((CH, d), jnp.float32)
    idx_t = pltpu.VMEM((CH,), jnp.int32)
    w_t = pltpu.VMEM((CH,), jnp.float32)
    sem_t = pltpu.SemaphoreType.DMA

    @functools.partial(
        pl.kernel,
        mesh=mesh,
        out_type=jax.ShapeDtypeStruct((2, n_nodes, d), jnp.float32),
        scratch_types=(
            [rows_t] * NBUF + [idx_t] * NBUF + [idx_t] * NBUF + [w_t] * NBUF
            + [pltpu.VMEM_SHARED((n_nodes, d), jnp.float32)]
            + [sem_t] * NBUF + [sem_t] * NBUF + [sem_t] * NBUF
        ),
    )
    def sc_agg(row_hbm, col_hbm, w_hbm, x_hbm, out_hbm, *scr):
        rows = scr[0:NBUF]
        col = scr[NBUF:2 * NBUF]
        dst = scr[2 * NBUF:3 * NBUF]
        wgt = scr[3 * NBUF:4 * NBUF]
        accum = scr[4 * NBUF]
        sem_i = scr[4 * NBUF + 1:4 * NBUF + 1 + NBUF]
        sem_g = scr[4 * NBUF + 1 + NBUF:4 * NBUF + 1 + 2 * NBUF]
        sem_s = scr[4 * NBUF + 1 + 2 * NBUF:4 * NBUF + 1 + 3 * NBUF]

        c = lax.axis_index("c")
        s = lax.axis_index("s")
        wid = s * 2 + c
        base = wid * CPW

        zero16 = jnp.zeros((LANES,), jnp.float32)

        def _zero_row(i, _):
            for j in range(vregs_per_row):
                rows[0][i, pl.ds(j * LANES, LANES)] = zero16
            return 0

        lax.fori_loop(0, CH, _zero_row, 0)
        for h in range(n_hops):
            pltpu.sync_copy(rows[0].at[pl.ds(0, hop)],
                            accum.at[pl.ds(s * rows_per_tile + h * hop, hop)])
        if tail:
            @pl.when(s == 15)
            def _():
                pltpu.sync_copy(rows[0].at[pl.ds(0, tail)],
                                accum.at[pl.ds(16 * rows_per_tile, tail)])
        plsc.subcore_barrier()

        def _idx_start(k, b):
            pltpu.async_copy(col_hbm.at[base + k], col[b], sem_i[b])
            pltpu.async_copy(row_hbm.at[base + k], dst[b], sem_i[b])
            pltpu.async_copy(w_hbm.at[base + k], wgt[b], sem_i[b])

        def _idx_wait(b):
            pltpu.make_async_copy(col_hbm.at[0], col[b], sem_i[b]).wait()
            pltpu.make_async_copy(row_hbm.at[0], dst[b], sem_i[b]).wait()
            pltpu.make_async_copy(w_hbm.at[0], wgt[b], sem_i[b]).wait()

        def _gather_start(b):
            pltpu.async_copy(x_hbm.at[col[b]], rows[b], sem_g[b])

        def _gather_wait(b):
            pltpu.make_async_copy(x_hbm.at[col[b]], rows[b], sem_g[b]).wait()

        def _scatter_start(b):
            pltpu.async_copy(rows[b], accum.at[dst[b]], sem_s[b], add=True)

        def _scatter_wait(b):
            pltpu.make_async_copy(rows[b], accum.at[dst[b]], sem_s[b]).wait()

        def _scale(b):
            def _group(g, _):
                w16 = wgt[b][pl.ds(g * LANES, LANES)]
                for lane in range(LANES):
                    e = g * LANES + lane
                    wvec = jnp.full((LANES,), w16[lane], jnp.float32)
                    for j in range(vregs_per_row):
                        rows[b][e, pl.ds(j * LANES, LANES)] = (
                            rows[b][e, pl.ds(j * LANES, LANES)] * wvec)
                return 0

            lax.fori_loop(0, CH // LANES, _group, 0)

        _idx_start(0, 0)
        _idx_start(1, 1)
        _idx_wait(0)
        _gather_start(0)

        def _step(k, b):
            bn = (b + 1) % NBUF
            bp = (b + 2) % NBUF

            @pl.when(k + 1 < CPW)
            def _():
                _idx_wait(bn)
                _gather_start(bn)

            _gather_wait(b)
            _scale(b)
            _scatter_start(b)

            @pl.when(k + 2 < CPW)
            def _():
                @pl.when(k >= 1)
                def _():
                    _scatter_wait(bp)
                _idx_start(k + 2, bp)

        def _body(g, _):
            for bb in range(NBUF):
                _step(g * NBUF + bb, bb)
            return 0

        lax.fori_loop(0, CPW // NBUF, _body, 0)
        for b in range(NBUF):
            _scatter_wait(b)
        plsc.subcore_barrier()

        for h in range(n_hops):
            r0 = s * rows_per_tile + h * hop
            pltpu.sync_copy(accum.at[pl.ds(r0, hop)], rows[0].at[pl.ds(0, hop)])
            pltpu.sync_copy(rows[0].at[pl.ds(0, hop)], out_hbm.at[c, pl.ds(r0, hop)])
        if tail:
            @pl.when(s == 15)
            def _():
                r0 = 16 * rows_per_tile
                pltpu.sync_copy(accum.at[pl.ds(r0, tail)], rows[0].at[pl.ds(0, tail)])
                pltpu.sync_copy(rows[0].at[pl.ds(0, tail)],
                                out_hbm.at[c, pl.ds(r0, tail)])

    return sc_agg


def _tc_body(x_ref, a_ref, ws_ref, wn_ref, b_ref, o_ref):
    xb = x_ref[...]
    ab = a_ref[0] + a_ref[1]
    dn = (((1,), (1,)), ((), ()))
    o_ref[...] = (
        lax.dot_general(xb, ws_ref[...], dn, preferred_element_type=jnp.float32)
        + lax.dot_general(ab, wn_ref[...], dn, preferred_element_type=jnp.float32)
        + b_ref[...]
    )


@functools.lru_cache(maxsize=None)
def _make_tc_linear(n_nodes: int, d: int):
    br = 1000
    assert n_nodes % br == 0
    grid = (n_nodes // br,)
    return pl.pallas_call(
        _tc_body,
        grid=grid,
        in_specs=[
            pl.BlockSpec((br, d), lambda i: (i, 0)),
            pl.BlockSpec((2, br, d), lambda i: (0, i, 0)),
            pl.BlockSpec((d, d), lambda i: (0, 0)),
            pl.BlockSpec((d, d), lambda i: (0, 0)),
            pl.BlockSpec((1, d), lambda i: (0, 0)),
        ],
        out_specs=pl.BlockSpec((br, d), lambda i: (i, 0)),
        out_shape=jax.ShapeDtypeStruct((n_nodes, d), jnp.float32),
    )


def kernel(x, edge_index, edge_weight, num_nodes, W_self, b_self, W_neigh, b_neigh):
    n, d = x.shape
    e = edge_index.shape[1]
    ei = edge_index.astype(jnp.int32)
    row = (ei[0] % jnp.asarray(num_nodes, jnp.int32)).astype(jnp.int32)
    col = ei[1]
    ep = NW * CPW * CH
    pad = ep - e
    assert pad >= 0
    row2d = jnp.concatenate([row, jnp.zeros((pad,), jnp.int32)]).reshape(-1, CH)
    col2d = jnp.concatenate([col, jnp.zeros((pad,), jnp.int32)]).reshape(-1, CH)
    w2d = jnp.concatenate(
        [edge_weight.astype(jnp.float32), jnp.zeros((pad,), jnp.float32)]
    ).reshape(-1, CH)
    agg = _make_sc_aggregate(n, d)(row2d, col2d, w2d, x)
    bias = (b_self + b_neigh).reshape(1, d).astype(jnp.float32)
    return _make_tc_linear(n, d)(x, agg, W_self, W_neigh, bias)


# speedup vs baseline: 1.2140x; 1.2140x over previous
import functools

import jax
import jax.numpy as jnp
from jax import lax
from jax.experimental import pallas as pl
from jax.experimental.pallas import tpu as pltpu
from jax.experimental.pallas import tpu_sc as plsc

CH = 128
LANES = 16
NW = 32
CPW = 80
NBUF = 2


@functools.lru_cache(maxsize=None)
def _make_sc_aggregate(n_nodes: int, d: int):
    rows_per_tile = (n_nodes // (16 * 8)) * 8
    hop = 104
    n_hops = rows_per_tile // hop
    assert n_hops * hop == rows_per_tile
    tail = n_nodes - 16 * rows_per_tile
    assert 0 <= tail <= CH and tail % 8 == 0
    vregs_per_row = d // LANES

    mesh = plsc.VectorSubcoreMesh(core_axis_name="c", subcore_axis_name="s")

    rows_t = pltpu.VMEM((CH, d), jnp.float32)
    idx_t = pltpu.VMEM((CH,), jnp.int32)
    w_t = pltpu.VMEM((CH,), jnp.float32)
    sem_t = pltpu.SemaphoreType.DMA

    @functools.partial(
        pl.kernel,
        mesh=mesh,
        out_type=jax.ShapeDtypeStruct((2, n_nodes, d), jnp.float32),
        scratch_types=(
            [rows_t] * NBUF + [idx_t] * NBUF + [idx_t] * NBUF + [w_t] * NBUF
            + [pltpu.VMEM_SHARED((n_nodes, d), jnp.float32)]
            + [sem_t] * NBUF + [sem_t] * NBUF + [sem_t] * NBUF
        ),
    )
    def sc_agg(row_hbm, col_hbm, w_hbm, x_hbm, out_hbm, *scr):
        rows = scr[0:NBUF]
        col = scr[NBUF:2 * NBUF]
        dst = scr[2 * NBUF:3 * NBUF]
        wgt = scr[3 * NBUF:4 * NBUF]
        accum = scr[4 * NBUF]
        sem_g = scr[4 * NBUF + 1:4 * NBUF + 1 + NBUF]

        c = lax.axis_index("c")
        s = lax.axis_index("s")
        wid = s * 2 + c
        base = wid * CPW

        zero16 = jnp.zeros((LANES,), jnp.float32)

        def _zero_row(i, _):
            for j in range(vregs_per_row):
                rows[0][i, pl.ds(j * LANES, LANES)] = zero16
            return 0

        lax.fori_loop(0, CH, _zero_row, 0)
        for h in range(n_hops):
            pltpu.sync_copy(rows[0].at[pl.ds(0, hop)],
                            accum.at[pl.ds(s * rows_per_tile + h * hop, hop)])
        if tail:
            @pl.when(s == 15)
            def _():
                pltpu.sync_copy(rows[0].at[pl.ds(0, tail)],
                                accum.at[pl.ds(16 * rows_per_tile, tail)])
        plsc.subcore_barrier()

        def _idx_load(k, b):
            pltpu.sync_copy(col_hbm.at[base + k], col[b])
            pltpu.sync_copy(row_hbm.at[base + k], dst[b])
            pltpu.sync_copy(w_hbm.at[base + k], wgt[b])

        def _gather_start(b):
            pltpu.async_copy(x_hbm.at[col[b]], rows[b], sem_g[b])

        def _gather_wait(b):
            pltpu.make_async_copy(x_hbm.at[col[b]], rows[b], sem_g[b]).wait()

        def _scatter(b):
            pltpu.sync_copy(rows[b], accum.at[dst[b]], add=True)

        def _scale(b):
            def _group(g, _):
                w16 = wgt[b][pl.ds(g * LANES, LANES)]
                for lane in range(LANES):
                    e = g * LANES + lane
                    wvec = jnp.full((LANES,), w16[lane], jnp.float32)
                    for j in range(vregs_per_row):
                        rows[b][e, pl.ds(j * LANES, LANES)] = (
                            rows[b][e, pl.ds(j * LANES, LANES)] * wvec)
                return 0

            lax.fori_loop(0, CH // LANES, _group, 0)

        _idx_load(0, 0)
        _gather_start(0)

        def _step(k, b):
            bn = (b + 1) % NBUF

            @pl.when(k + 1 < CPW)
            def _():
                _idx_load(k + 1, bn)
                _gather_start(bn)

            _gather_wait(b)
            _scale(b)
            _scatter(b)

        def _body(g, _):
            for bb in range(NBUF):
                _step(g * NBUF + bb, bb)
            return 0

        lax.fori_loop(0, CPW // NBUF, _body, 0)
        plsc.subcore_barrier()

        for h in range(n_hops):
            r0 = s * rows_per_tile + h * hop
            pltpu.sync_copy(accum.at[pl.ds(r0, hop)], rows[0].at[pl.ds(0, hop)])
            pltpu.sync_copy(rows[0].at[pl.ds(0, hop)], out_hbm.at[c, pl.ds(r0, hop)])
        if tail:
            @pl.when(s == 15)
            def _():
                r0 = 16 * rows_per_tile
                pltpu.sync_copy(accum.at[pl.ds(r0, tail)], rows[0].at[pl.ds(0, tail)])
                pltpu.sync_copy(rows[0].at[pl.ds(0, tail)],
                                out_hbm.at[c, pl.ds(r0, tail)])

    return sc_agg


def _tc_body(x_ref, a_ref, ws_ref, wn_ref, b_ref, o_ref):
    xb = x_ref[...]
    ab = a_ref[0] + a_ref[1]
    dn = (((1,), (1,)), ((), ()))
    o_ref[...] = (
        lax.dot_general(xb, ws_ref[...], dn, preferred_element_type=jnp.float32)
        + lax.dot_general(ab, wn_ref[...], dn, preferred_element_type=jnp.float32)
        + b_ref[...]
    )


@functools.lru_cache(maxsize=None)
def _make_tc_linear(n_nodes: int, d: int):
    br = 1000
    assert n_nodes % br == 0
    grid = (n_nodes // br,)
    return pl.pallas_call(
        _tc_body,
        grid=grid,
        in_specs=[
            pl.BlockSpec((br, d), lambda i: (i, 0)),
            pl.BlockSpec((2, br, d), lambda i: (0, i, 0)),
            pl.BlockSpec((d, d), lambda i: (0, 0)),
            pl.BlockSpec((d, d), lambda i: (0, 0)),
            pl.BlockSpec((1, d), lambda i: (0, 0)),
        ],
        out_specs=pl.BlockSpec((br, d), lambda i: (i, 0)),
        out_shape=jax.ShapeDtypeStruct((n_nodes, d), jnp.float32),
    )


def kernel(x, edge_index, edge_weight, num_nodes, W_self, b_self, W_neigh, b_neigh):
    n, d = x.shape
    e = edge_index.shape[1]
    ei = edge_index.astype(jnp.int32)
    row = (ei[0] % jnp.asarray(num_nodes, jnp.int32)).astype(jnp.int32)
    col = ei[1]
    ep = NW * CPW * CH
    pad = ep - e
    assert pad >= 0
    row2d = jnp.concatenate([row, jnp.zeros((pad,), jnp.int32)]).reshape(-1, CH)
    col2d = jnp.concatenate([col, jnp.zeros((pad,), jnp.int32)]).reshape(-1, CH)
    w2d = jnp.concatenate(
        [edge_weight.astype(jnp.float32), jnp.zeros((pad,), jnp.float32)]
    ).reshape(-1, CH)
    agg = _make_sc_aggregate(n, d)(row2d, col2d, w2d, x)
    bias = (b_self + b_neigh).reshape(1, d).astype(jnp.float32)
    return _make_tc_linear(n, d)(x, agg, W_self, W_neigh, bias)
